# baseline (speedup 1.0000x reference)
import jax
import jax.numpy as jnp
from jax.experimental import pallas as pl
from jax.experimental.pallas import tpu as pltpu

ALPHA = 1.702
LIMIT = 7.0
EPS = 1e-05
NEG = -1e30


def _router_body(x_ref, scale_ref, gw_ref, gb_ref,
                 t_ref, ct_ref, sched_ref, n_ref):
    x = x_ref[...]
    ms = jnp.mean(x * x, axis=1, keepdims=True)
    t = x * jax.lax.rsqrt(ms + EPS) * scale_ref[...]
    t_ref[...] = t
    g = jax.lax.dot_general(t, gw_ref[...], (((1,), (1,)), ((), ())),
                            preferred_element_type=jnp.float32,
                            precision=jax.lax.Precision.HIGHEST)
    g = g + gb_ref[...]
    n_tok, n_exp = g.shape
    iota_e = jax.lax.broadcasted_iota(jnp.int32, (n_tok, n_exp), 1)
    v1 = jnp.max(g, axis=1, keepdims=True)
    idx1 = jnp.min(jnp.where(g == v1, iota_e, n_exp), axis=1, keepdims=True)
    e1 = iota_e == idx1
    g2 = jnp.where(e1, NEG, g)
    v2 = jnp.max(g2, axis=1, keepdims=True)
    idx2 = jnp.min(jnp.where(g2 == v2, iota_e, n_exp), axis=1, keepdims=True)
    e2 = iota_e == idx2
    p1 = jax.nn.sigmoid(v1 - v2)
    p2 = 1.0 - p1
    ct_ref[...] = p1 * e1.astype(jnp.float32) + p2 * e2.astype(jnp.float32)
    hits = jnp.transpose((e1 | e2).astype(jnp.float32))
    cnts = jnp.sum(hits, axis=1, keepdims=True)
    m = (cnts > 0.0).astype(jnp.float32)
    row = jax.lax.broadcasted_iota(jnp.int32, (n_exp, n_exp), 0).astype(jnp.float32)
    col = jax.lax.broadcasted_iota(jnp.int32, (n_exp, n_exp), 1).astype(jnp.float32)
    tril = (row >= col).astype(jnp.float32)
    c = jax.lax.dot_general(tril, m, (((1,), (0,)), ((), ())),
                            preferred_element_type=jnp.float32)
    onehot = m * (c - 1.0 == col).astype(jnp.float32)
    sched = jnp.sum(row * onehot, axis=0, keepdims=True)
    sched_ref[...] = sched.astype(jnp.int32)
    n_ref[...] = jnp.sum(m, keepdims=True).astype(jnp.int32)


def _ffn_body(sched_ref, n_ref, t_ref, x_ref, ct_ref,
              w1_ref, b1_ref, w2_ref, b2_ref, out_ref):
    i = pl.program_id(0)
    n = n_ref[0]

    @pl.when(i == 0)
    def _init():
        out_ref[...] = x_ref[...]

    @pl.when(i < n)
    def _compute():
        e = sched_ref[i]
        t = t_ref[...]
        w1 = w1_ref[:, 0:t.shape[1]]
        h = jax.lax.dot_general(t, w1, (((1,), (1,)), ((), ())),
                                preferred_element_type=jnp.float32)
        h = h + b1_ref[...]
        a = jnp.minimum(h, LIMIT)
        s = (a * jax.nn.sigmoid(ALPHA * a)) * (LIMIT + 1.0)
        h2 = jax.lax.dot_general(s, w2_ref[...], (((1,), (1,)), ((), ())),
                                 preferred_element_type=jnp.float32)
        h2 = h2 + b2_ref[...]
        ct = ct_ref[...]
        lane = jax.lax.broadcasted_iota(jnp.int32, ct.shape, 1)
        cvec = jnp.sum(jnp.where(lane == e, ct, 0.0), axis=1, keepdims=True)
        out_ref[...] += cvec * h2


def kernel(x, scale, gate_w, gate_b, mlp1_w, mlp1_b, mlp2_w, mlp2_b):
    B, T, H = x.shape
    N = B * T
    E, F2, _ = mlp1_w.shape
    F = F2 // 2
    xt = x.reshape(N, H).astype(jnp.float32)

    t_norm, ct, sched, nact = pl.pallas_call(
        _router_body,
        out_shape=(
            jax.ShapeDtypeStruct((N, H), jnp.float32),
            jax.ShapeDtypeStruct((N, E), jnp.float32),
            jax.ShapeDtypeStruct((1, E), jnp.int32),
            jax.ShapeDtypeStruct((1, 1), jnp.int32),
        ),
    )(xt, scale.reshape(1, H), gate_w, gate_b.reshape(1, E))

    w1r = mlp1_w.reshape(E, F, 2 * H)
    b1g = mlp1_b[:, 0::2].reshape(E, 1, F)
    b2r = mlp2_b.reshape(E, 1, H)

    grid_spec = pltpu.PrefetchScalarGridSpec(
        num_scalar_prefetch=2,
        grid=(E,),
        in_specs=[
            pl.BlockSpec((N, H), lambda i, s, n: (0, 0)),
            pl.BlockSpec((N, H), lambda i, s, n: (0, 0)),
            pl.BlockSpec((N, E), lambda i, s, n: (0, 0)),
            pl.BlockSpec((None, F, 2 * H),
                         lambda i, s, n: (s[jnp.minimum(i, n[0] - 1)], 0, 0)),
            pl.BlockSpec((None, 1, F),
                         lambda i, s, n: (s[jnp.minimum(i, n[0] - 1)], 0, 0)),
            pl.BlockSpec((None, H, F),
                         lambda i, s, n: (s[jnp.minimum(i, n[0] - 1)], 0, 0)),
            pl.BlockSpec((None, 1, H),
                         lambda i, s, n: (s[jnp.minimum(i, n[0] - 1)], 0, 0)),
        ],
        out_specs=pl.BlockSpec((N, H), lambda i, s, n: (0, 0)),
    )

    out = pl.pallas_call(
        _ffn_body,
        grid_spec=grid_spec,
        out_shape=jax.ShapeDtypeStruct((N, H), jnp.float32),
        compiler_params=pltpu.CompilerParams(
            dimension_semantics=("arbitrary",)),
    )(sched.reshape(E), nact.reshape(1), t_norm, xt, ct,
      w1r, b1g, mlp2_w, b2r)

    return out.reshape(B, T, H).astype(x.dtype)


# speedup vs baseline: 3.3318x; 3.3318x over previous
import jax
import jax.numpy as jnp
from jax.experimental import pallas as pl
from jax.experimental.pallas import tpu as pltpu

ALPHA = 1.702
LIMIT = 7.0
EPS = 1e-05
NEG = -1e30


def _router_body(x_ref, scale_ref, gw_ref, gb_ref,
                 t_ref, ct_ref, sched_ref, n_ref):
    x = x_ref[...]
    ms = jnp.mean(x * x, axis=1, keepdims=True)
    t = x * jax.lax.rsqrt(ms + EPS) * scale_ref[...]
    t_ref[...] = t
    g = jax.lax.dot_general(t, gw_ref[...], (((1,), (1,)), ((), ())),
                            preferred_element_type=jnp.float32,
                            precision=jax.lax.Precision.HIGHEST)
    g = g + gb_ref[...]
    n_tok, n_exp = g.shape
    iota_e = jax.lax.broadcasted_iota(jnp.int32, (n_tok, n_exp), 1)
    v1 = jnp.max(g, axis=1, keepdims=True)
    idx1 = jnp.min(jnp.where(g == v1, iota_e, n_exp), axis=1, keepdims=True)
    e1 = iota_e == idx1
    g2 = jnp.where(e1, NEG, g)
    v2 = jnp.max(g2, axis=1, keepdims=True)
    idx2 = jnp.min(jnp.where(g2 == v2, iota_e, n_exp), axis=1, keepdims=True)
    e2 = iota_e == idx2
    p1 = jax.nn.sigmoid(v1 - v2)
    p2 = 1.0 - p1
    ct_ref[...] = p1 * e1.astype(jnp.float32) + p2 * e2.astype(jnp.float32)
    hits = jnp.transpose((e1 | e2).astype(jnp.float32))
    cnts = jnp.sum(hits, axis=1, keepdims=True)
    m = (cnts > 0.0).astype(jnp.float32)
    row = jax.lax.broadcasted_iota(jnp.int32, (n_exp, n_exp), 0).astype(jnp.float32)
    col = jax.lax.broadcasted_iota(jnp.int32, (n_exp, n_exp), 1).astype(jnp.float32)
    tril = (row >= col).astype(jnp.float32)
    c = jax.lax.dot_general(tril, m, (((1,), (0,)), ((), ())),
                            preferred_element_type=jnp.float32)
    onehot = m * (c - 1.0 == col).astype(jnp.float32)
    sched = jnp.sum(row * onehot, axis=0, keepdims=True)
    sched_ref[...] = sched.astype(jnp.int32)
    n_ref[...] = jnp.sum(m, keepdims=True).astype(jnp.int32)


def _ffn_body(sched_ref, n_ref, t_ref, x_ref, ct_ref,
              w1_ref, b1_ref, w2_ref, b2_ref, out_ref, p_ref):
    i = pl.program_id(0)
    n = n_ref[0]

    @pl.when(i == 0)
    def _init():
        out_ref[...] = x_ref[...]
        f2, f = p_ref.shape
        rr = jax.lax.broadcasted_iota(jnp.int32, (f2, f), 0)
        cc = jax.lax.broadcasted_iota(jnp.int32, (f2, f), 1)
        p_ref[...] = (rr == 2 * cc).astype(jnp.float32)

    @pl.when(i < n)
    def _compute():
        e = sched_ref[i]
        t = t_ref[...]
        h = jax.lax.dot_general(t, w1_ref[...], (((1,), (1,)), ((), ())),
                                preferred_element_type=jnp.float32)
        h = h + b1_ref[...]
        a = jnp.minimum(h, LIMIT)
        s = (a * jax.nn.sigmoid(ALPHA * a)) * (LIMIT + 1.0)
        sc = jax.lax.dot_general(s, p_ref[...], (((1,), (0,)), ((), ())),
                                 preferred_element_type=jnp.float32)
        h2 = jax.lax.dot_general(sc, w2_ref[...], (((1,), (1,)), ((), ())),
                                 preferred_element_type=jnp.float32)
        h2 = h2 + b2_ref[...]
        ct = ct_ref[...]
        lane = jax.lax.broadcasted_iota(jnp.int32, ct.shape, 1)
        cvec = jnp.sum(jnp.where(lane == e, ct, 0.0), axis=1, keepdims=True)
        out_ref[...] += cvec * h2


def kernel(x, scale, gate_w, gate_b, mlp1_w, mlp1_b, mlp2_w, mlp2_b):
    B, T, H = x.shape
    N = B * T
    E, F2, _ = mlp1_w.shape
    F = F2 // 2
    xt = x.reshape(N, H).astype(jnp.float32)

    t_norm, ct, sched, nact = pl.pallas_call(
        _router_body,
        out_shape=(
            jax.ShapeDtypeStruct((N, H), jnp.float32),
            jax.ShapeDtypeStruct((N, E), jnp.float32),
            jax.ShapeDtypeStruct((1, E), jnp.int32),
            jax.ShapeDtypeStruct((1, 1), jnp.int32),
        ),
    )(xt, scale.reshape(1, H), gate_w, gate_b.reshape(1, E))

    b1r = mlp1_b.reshape(E, 1, F2)
    b2r = mlp2_b.reshape(E, 1, H)

    grid_spec = pltpu.PrefetchScalarGridSpec(
        num_scalar_prefetch=2,
        grid=(E,),
        in_specs=[
            pl.BlockSpec((N, H), lambda i, s, n: (0, 0)),
            pl.BlockSpec((N, H), lambda i, s, n: (0, 0)),
            pl.BlockSpec((N, E), lambda i, s, n: (0, 0)),
            pl.BlockSpec((None, F2, H),
                         lambda i, s, n: (s[jnp.minimum(i, n[0] - 1)], 0, 0)),
            pl.BlockSpec((None, 1, F2),
                         lambda i, s, n: (s[jnp.minimum(i, n[0] - 1)], 0, 0)),
            pl.BlockSpec((None, H, F),
                         lambda i, s, n: (s[jnp.minimum(i, n[0] - 1)], 0, 0)),
            pl.BlockSpec((None, 1, H),
                         lambda i, s, n: (s[jnp.minimum(i, n[0] - 1)], 0, 0)),
        ],
        out_specs=pl.BlockSpec((N, H), lambda i, s, n: (0, 0)),
        scratch_shapes=[pltpu.VMEM((F2, F), jnp.float32)],
    )

    out = pl.pallas_call(
        _ffn_body,
        grid_spec=grid_spec,
        out_shape=jax.ShapeDtypeStruct((N, H), jnp.float32),
        compiler_params=pltpu.CompilerParams(
            dimension_semantics=("arbitrary",)),
    )(sched.reshape(E), nact.reshape(1), t_norm, xt, ct,
      mlp1_w, b1r, mlp2_w, b2r)

    return out.reshape(B, T, H).astype(x.dtype)


# speedup vs baseline: 3.3365x; 1.0014x over previous
import jax
import jax.numpy as jnp
from jax.experimental import pallas as pl
from jax.experimental.pallas import tpu as pltpu

ALPHA = 1.702
LIMIT = 7.0
EPS = 1e-05
NEG = -1e30


def _router_body(x_ref, scale_ref, gw_ref, gb_ref,
                 t_ref, ct_ref, sched_ref, n_ref):
    x = x_ref[...]
    ms = jnp.mean(x * x, axis=1, keepdims=True)
    t = x * jax.lax.rsqrt(ms + EPS) * scale_ref[...]
    t_ref[...] = t
    g = jax.lax.dot_general(t, gw_ref[...], (((1,), (1,)), ((), ())),
                            preferred_element_type=jnp.float32)
    g = g + gb_ref[...]
    n_tok, n_exp = g.shape
    iota_e = jax.lax.broadcasted_iota(jnp.int32, (n_tok, n_exp), 1)
    v1 = jnp.max(g, axis=1, keepdims=True)
    idx1 = jnp.min(jnp.where(g == v1, iota_e, n_exp), axis=1, keepdims=True)
    e1 = iota_e == idx1
    g2 = jnp.where(e1, NEG, g)
    v2 = jnp.max(g2, axis=1, keepdims=True)
    idx2 = jnp.min(jnp.where(g2 == v2, iota_e, n_exp), axis=1, keepdims=True)
    e2 = iota_e == idx2
    p1 = jax.nn.sigmoid(v1 - v2)
    p2 = 1.0 - p1
    ct_ref[...] = p1 * e1.astype(jnp.float32) + p2 * e2.astype(jnp.float32)
    hits = jnp.transpose((e1 | e2).astype(jnp.float32))
    cnts = jnp.sum(hits, axis=1, keepdims=True)
    m = (cnts > 0.0).astype(jnp.float32)
    row = jax.lax.broadcasted_iota(jnp.int32, (n_exp, n_exp), 0).astype(jnp.float32)
    col = jax.lax.broadcasted_iota(jnp.int32, (n_exp, n_exp), 1).astype(jnp.float32)
    tril = (row >= col).astype(jnp.float32)
    c = jax.lax.dot_general(tril, m, (((1,), (0,)), ((), ())),
                            preferred_element_type=jnp.float32)
    onehot = m * (c - 1.0 == col).astype(jnp.float32)
    sched = jnp.sum(row * onehot, axis=0, keepdims=True)
    sched_ref[...] = sched.astype(jnp.int32)
    n_ref[...] = jnp.sum(m, keepdims=True).astype(jnp.int32)


def _ffn_body(sched_ref, n_ref, t_ref, x_ref, ct_ref,
              w1_ref, b1_ref, w2_ref, b2_ref, out_ref, p_ref):
    i = pl.program_id(0)
    n = n_ref[0]

    @pl.when(i == 0)
    def _init():
        out_ref[...] = x_ref[...]
        f2, f = p_ref.shape
        rr = jax.lax.broadcasted_iota(jnp.int32, (f2, f), 0)
        cc = jax.lax.broadcasted_iota(jnp.int32, (f2, f), 1)
        p_ref[...] = (rr == 2 * cc).astype(jnp.float32)

    @pl.when(i < n)
    def _compute():
        e = sched_ref[i]
        t = t_ref[...]
        h = jax.lax.dot_general(t, w1_ref[...], (((1,), (1,)), ((), ())),
                                preferred_element_type=jnp.float32)
        h = h + b1_ref[...]
        a = jnp.minimum(h, LIMIT)
        s = (a * jax.nn.sigmoid(ALPHA * a)) * (LIMIT + 1.0)
        sc = jax.lax.dot_general(s, p_ref[...], (((1,), (0,)), ((), ())),
                                 preferred_element_type=jnp.float32)
        h2 = jax.lax.dot_general(sc, w2_ref[...], (((1,), (1,)), ((), ())),
                                 preferred_element_type=jnp.float32)
        h2 = h2 + b2_ref[...]
        ct = ct_ref[...]
        lane = jax.lax.broadcasted_iota(jnp.int32, ct.shape, 1)
        cvec = jnp.sum(jnp.where(lane == e, ct, 0.0), axis=1, keepdims=True)
        out_ref[...] += cvec * h2


def kernel(x, scale, gate_w, gate_b, mlp1_w, mlp1_b, mlp2_w, mlp2_b):
    B, T, H = x.shape
    N = B * T
    E, F2, _ = mlp1_w.shape
    F = F2 // 2
    xt = x.reshape(N, H).astype(jnp.float32)

    t_norm, ct, sched, nact = pl.pallas_call(
        _router_body,
        out_shape=(
            jax.ShapeDtypeStruct((N, H), jnp.float32),
            jax.ShapeDtypeStruct((N, E), jnp.float32),
            jax.ShapeDtypeStruct((1, E), jnp.int32),
            jax.ShapeDtypeStruct((1, 1), jnp.int32),
        ),
    )(xt, scale.reshape(1, H), gate_w, gate_b.reshape(1, E))

    b1r = mlp1_b.reshape(E, 1, F2)
    b2r = mlp2_b.reshape(E, 1, H)

    grid_spec = pltpu.PrefetchScalarGridSpec(
        num_scalar_prefetch=2,
        grid=(E,),
        in_specs=[
            pl.BlockSpec((N, H), lambda i, s, n: (0, 0)),
            pl.BlockSpec((N, H), lambda i, s, n: (0, 0)),
            pl.BlockSpec((N, E), lambda i, s, n: (0, 0)),
            pl.BlockSpec((None, F2, H),
                         lambda i, s, n: (s[jnp.minimum(i, n[0] - 1)], 0, 0)),
            pl.BlockSpec((None, 1, F2),
                         lambda i, s, n: (s[jnp.minimum(i, n[0] - 1)], 0, 0)),
            pl.BlockSpec((None, H, F),
                         lambda i, s, n: (s[jnp.minimum(i, n[0] - 1)], 0, 0)),
            pl.BlockSpec((None, 1, H),
                         lambda i, s, n: (s[jnp.minimum(i, n[0] - 1)], 0, 0)),
        ],
        out_specs=pl.BlockSpec((N, H), lambda i, s, n: (0, 0)),
        scratch_shapes=[pltpu.VMEM((F2, F), jnp.float32)],
    )

    out = pl.pallas_call(
        _ffn_body,
        grid_spec=grid_spec,
        out_shape=jax.ShapeDtypeStruct((N, H), jnp.float32),
        compiler_params=pltpu.CompilerParams(
            dimension_semantics=("arbitrary",)),
    )(sched.reshape(E), nact.reshape(1), t_norm, xt, ct,
      mlp1_w, b1r, mlp2_w, b2r)

    return out.reshape(B, T, H).astype(x.dtype)


# speedup vs baseline: 3.8247x; 1.1463x over previous
import jax
import jax.numpy as jnp
from jax.experimental import pallas as pl
from jax.experimental.pallas import tpu as pltpu

ALPHA = 1.702
LIMIT = 7.0
EPS = 1e-05
NEG = -1e30


def _router_body(x_ref, scale_ref, gw_ref, gb_ref,
                 t_ref, ct_ref, sched_ref, n_ref):
    x = x_ref[...]
    ms = jnp.mean(x * x, axis=1, keepdims=True)
    t = x * jax.lax.rsqrt(ms + EPS) * scale_ref[...]
    t_ref[...] = t
    g = jax.lax.dot_general(t, gw_ref[...], (((1,), (1,)), ((), ())),
                            preferred_element_type=jnp.float32)
    g = g + gb_ref[...]
    n_tok, n_exp = g.shape
    iota_e = jax.lax.broadcasted_iota(jnp.int32, (n_tok, n_exp), 1)
    v1 = jnp.max(g, axis=1, keepdims=True)
    idx1 = jnp.min(jnp.where(g == v1, iota_e, n_exp), axis=1, keepdims=True)
    e1 = iota_e == idx1
    g2 = jnp.where(e1, NEG, g)
    v2 = jnp.max(g2, axis=1, keepdims=True)
    idx2 = jnp.min(jnp.where(g2 == v2, iota_e, n_exp), axis=1, keepdims=True)
    e2 = iota_e == idx2
    p1 = jax.nn.sigmoid(v1 - v2)
    p2 = 1.0 - p1
    ct_ref[...] = p1 * e1.astype(jnp.float32) + p2 * e2.astype(jnp.float32)
    hits = jnp.transpose((e1 | e2).astype(jnp.float32))
    cnts = jnp.sum(hits, axis=1, keepdims=True)
    m = (cnts > 0.0).astype(jnp.float32)
    row = jax.lax.broadcasted_iota(jnp.int32, (n_exp, n_exp), 0).astype(jnp.float32)
    col = jax.lax.broadcasted_iota(jnp.int32, (n_exp, n_exp), 1).astype(jnp.float32)
    tril = (row >= col).astype(jnp.float32)
    c = jax.lax.dot_general(tril, m, (((1,), (0,)), ((), ())),
                            preferred_element_type=jnp.float32)
    onehot = m * (c - 1.0 == col).astype(jnp.float32)
    sched = jnp.sum(row * onehot, axis=0, keepdims=True)
    sched_ref[...] = sched.astype(jnp.int32)
    n_ref[...] = jnp.sum(m, keepdims=True).astype(jnp.int32)


def _expert_contrib(t, ct, valid, e, w1_ref, b1_ref, w2_ref, b2_ref, p_ref):
    h = jax.lax.dot_general(t, w1_ref[...], (((1,), (1,)), ((), ())),
                            preferred_element_type=jnp.float32)
    h = h + b1_ref[...]
    a = jnp.minimum(h, LIMIT)
    s = (a * jax.nn.sigmoid(ALPHA * a)) * (LIMIT + 1.0)
    sc = jax.lax.dot_general(s, p_ref[...], (((1,), (0,)), ((), ())),
                             preferred_element_type=jnp.float32)
    h2 = jax.lax.dot_general(sc, w2_ref[...], (((1,), (1,)), ((), ())),
                             preferred_element_type=jnp.float32)
    h2 = h2 + b2_ref[...]
    lane = jax.lax.broadcasted_iota(jnp.int32, ct.shape, 1)
    keep = jnp.logical_and(lane == e, valid)
    cvec = jnp.sum(jnp.where(keep, ct, 0.0), axis=1, keepdims=True)
    return cvec * h2


def _ffn_body(sched_ref, n_ref, t_ref, x_ref, ct_ref,
              w1a_ref, b1a_ref, w2a_ref, b2a_ref,
              w1b_ref, b1b_ref, w2b_ref, b2b_ref, out_ref, p_ref):
    i = pl.program_id(0)
    n = n_ref[0]

    @pl.when(i == 0)
    def _init():
        out_ref[...] = x_ref[...]
        f2, f = p_ref.shape
        rr = jax.lax.broadcasted_iota(jnp.int32, (f2, f), 0)
        cc = jax.lax.broadcasted_iota(jnp.int32, (f2, f), 1)
        p_ref[...] = (rr == 2 * cc).astype(jnp.float32)

    @pl.when(2 * i < n)
    def _compute():
        t = t_ref[...]
        ct = ct_ref[...]
        e_a = sched_ref[2 * i]
        e_b = sched_ref[jnp.minimum(2 * i + 1, n - 1)]
        valid_b = 2 * i + 1 < n
        da = _expert_contrib(t, ct, True, e_a,
                             w1a_ref, b1a_ref, w2a_ref, b2a_ref, p_ref)
        db = _expert_contrib(t, ct, valid_b, e_b,
                             w1b_ref, b1b_ref, w2b_ref, b2b_ref, p_ref)
        out_ref[...] += da + db


def kernel(x, scale, gate_w, gate_b, mlp1_w, mlp1_b, mlp2_w, mlp2_b):
    B, T, H = x.shape
    N = B * T
    E, F2, _ = mlp1_w.shape
    F = F2 // 2
    xt = x.reshape(N, H).astype(jnp.float32)

    t_norm, ct, sched, nact = pl.pallas_call(
        _router_body,
        out_shape=(
            jax.ShapeDtypeStruct((N, H), jnp.float32),
            jax.ShapeDtypeStruct((N, E), jnp.float32),
            jax.ShapeDtypeStruct((1, E), jnp.int32),
            jax.ShapeDtypeStruct((1, 1), jnp.int32),
        ),
    )(xt, scale.reshape(1, H), gate_w, gate_b.reshape(1, E))

    b1r = mlp1_b.reshape(E, 1, F2)
    b2r = mlp2_b.reshape(E, 1, H)

    def _slot(off):
        def _map(i, s, n):
            return (s[jnp.minimum(2 * i + off, n[0] - 1)], 0, 0)
        return _map

    grid_spec = pltpu.PrefetchScalarGridSpec(
        num_scalar_prefetch=2,
        grid=(E // 2,),
        in_specs=[
            pl.BlockSpec((N, H), lambda i, s, n: (0, 0)),
            pl.BlockSpec((N, H), lambda i, s, n: (0, 0)),
            pl.BlockSpec((N, E), lambda i, s, n: (0, 0)),
            pl.BlockSpec((None, F2, H), _slot(0)),
            pl.BlockSpec((None, 1, F2), _slot(0)),
            pl.BlockSpec((None, H, F), _slot(0)),
            pl.BlockSpec((None, 1, H), _slot(0)),
            pl.BlockSpec((None, F2, H), _slot(1)),
            pl.BlockSpec((None, 1, F2), _slot(1)),
            pl.BlockSpec((None, H, F), _slot(1)),
            pl.BlockSpec((None, 1, H), _slot(1)),
        ],
        out_specs=pl.BlockSpec((N, H), lambda i, s, n: (0, 0)),
        scratch_shapes=[pltpu.VMEM((F2, F), jnp.float32)],
    )

    out = pl.pallas_call(
        _ffn_body,
        grid_spec=grid_spec,
        out_shape=jax.ShapeDtypeStruct((N, H), jnp.float32),
        compiler_params=pltpu.CompilerParams(
            dimension_semantics=("arbitrary",)),
    )(sched.reshape(E), nact.reshape(1), t_norm, xt, ct,
      mlp1_w, b1r, mlp2_w, b2r,
      mlp1_w, b1r, mlp2_w, b2r)

    return out.reshape(B, T, H).astype(x.dtype)


# speedup vs baseline: 3.9052x; 1.0211x over previous
import jax
import jax.numpy as jnp
from jax.experimental import pallas as pl
from jax.experimental.pallas import tpu as pltpu

ALPHA = 1.702
LIMIT = 7.0
EPS = 1e-05
NEG = -1e30
SLOTS = 4


def _router_body(x_ref, scale_ref, gw_ref, gb_ref,
                 t_ref, ct_ref, sched_ref, n_ref):
    x = x_ref[...]
    ms = jnp.mean(x * x, axis=1, keepdims=True)
    t = x * jax.lax.rsqrt(ms + EPS) * scale_ref[...]
    t_ref[...] = t
    g = jax.lax.dot_general(t, gw_ref[...], (((1,), (1,)), ((), ())),
                            preferred_element_type=jnp.float32)
    g = g + gb_ref[...]
    n_tok, n_exp = g.shape
    iota_e = jax.lax.broadcasted_iota(jnp.int32, (n_tok, n_exp), 1)
    v1 = jnp.max(g, axis=1, keepdims=True)
    idx1 = jnp.min(jnp.where(g == v1, iota_e, n_exp), axis=1, keepdims=True)
    e1 = iota_e == idx1
    g2 = jnp.where(e1, NEG, g)
    v2 = jnp.max(g2, axis=1, keepdims=True)
    idx2 = jnp.min(jnp.where(g2 == v2, iota_e, n_exp), axis=1, keepdims=True)
    e2 = iota_e == idx2
    p1 = jax.nn.sigmoid(v1 - v2)
    p2 = 1.0 - p1
    ct_ref[...] = p1 * e1.astype(jnp.float32) + p2 * e2.astype(jnp.float32)
    hits = jnp.transpose((e1 | e2).astype(jnp.float32))
    cnts = jnp.sum(hits, axis=1, keepdims=True)
    m = (cnts > 0.0).astype(jnp.float32)
    row = jax.lax.broadcasted_iota(jnp.int32, (n_exp, n_exp), 0).astype(jnp.float32)
    col = jax.lax.broadcasted_iota(jnp.int32, (n_exp, n_exp), 1).astype(jnp.float32)
    tril = (row >= col).astype(jnp.float32)
    c = jax.lax.dot_general(tril, m, (((1,), (0,)), ((), ())),
                            preferred_element_type=jnp.float32)
    onehot = m * (c - 1.0 == col).astype(jnp.float32)
    sched = jnp.sum(row * onehot, axis=0, keepdims=True)
    sched_ref[...] = sched.astype(jnp.int32)
    n_ref[...] = jnp.sum(m, keepdims=True).astype(jnp.int32)


def _expert_contrib(t, ct, valid, e, w1_ref, w2_ref, b1_ref, b2_ref, p_ref):
    h = jax.lax.dot_general(t, w1_ref[...], (((1,), (1,)), ((), ())),
                            preferred_element_type=jnp.float32)
    h = h + b1_ref[pl.ds(e, 1), :]
    a = jnp.minimum(h, LIMIT)
    s = (a * jax.nn.sigmoid(ALPHA * a)) * (LIMIT + 1.0)
    sc = jax.lax.dot_general(s, p_ref[...], (((1,), (0,)), ((), ())),
                             preferred_element_type=jnp.float32)
    h2 = jax.lax.dot_general(sc, w2_ref[...], (((1,), (1,)), ((), ())),
                             preferred_element_type=jnp.float32)
    h2 = h2 + b2_ref[pl.ds(e, 1), :]
    lane = jax.lax.broadcasted_iota(jnp.int32, ct.shape, 1)
    keep = jnp.logical_and(lane == e, valid)
    cvec = jnp.sum(jnp.where(keep, ct, 0.0), axis=1, keepdims=True)
    return cvec * h2


def _ffn_body(sched_ref, n_ref, t_ref, x_ref, ct_ref, b1_ref, b2_ref,
              w1a_ref, w2a_ref, w1b_ref, w2b_ref,
              w1c_ref, w2c_ref, w1d_ref, w2d_ref, out_ref, p_ref):
    i = pl.program_id(0)
    n = n_ref[0]

    @pl.when(i == 0)
    def _init():
        out_ref[...] = x_ref[...]
        f2, f = p_ref.shape
        rr = jax.lax.broadcasted_iota(jnp.int32, (f2, f), 0)
        cc = jax.lax.broadcasted_iota(jnp.int32, (f2, f), 1)
        p_ref[...] = (rr == 2 * cc).astype(jnp.float32)

    @pl.when(SLOTS * i < n)
    def _compute():
        t = t_ref[...]
        ct = ct_ref[...]
        ws = [(w1a_ref, w2a_ref), (w1b_ref, w2b_ref),
              (w1c_ref, w2c_ref), (w1d_ref, w2d_ref)]
        acc = out_ref[...]
        for j, (w1_ref, w2_ref) in enumerate(ws):
            p = SLOTS * i + j
            e = sched_ref[jnp.minimum(p, n - 1)]
            valid = p < n if j else True
            acc = acc + _expert_contrib(t, ct, valid, e, w1_ref, w2_ref,
                                        b1_ref, b2_ref, p_ref)
        out_ref[...] = acc


def kernel(x, scale, gate_w, gate_b, mlp1_w, mlp1_b, mlp2_w, mlp2_b):
    B, T, H = x.shape
    N = B * T
    E, F2, _ = mlp1_w.shape
    F = F2 // 2
    xt = x.reshape(N, H).astype(jnp.float32)

    t_norm, ct, sched, nact = pl.pallas_call(
        _router_body,
        out_shape=(
            jax.ShapeDtypeStruct((N, H), jnp.float32),
            jax.ShapeDtypeStruct((N, E), jnp.float32),
            jax.ShapeDtypeStruct((1, E), jnp.int32),
            jax.ShapeDtypeStruct((1, 1), jnp.int32),
        ),
    )(xt, scale.reshape(1, H), gate_w, gate_b.reshape(1, E))

    def _slot(off):
        def _map(i, s, n):
            return (s[jnp.minimum(SLOTS * i + off, n[0] - 1)], 0, 0)
        return _map

    const2 = lambda i, s, n: (0, 0)
    w_specs = []
    for off in range(SLOTS):
        w_specs.append(pl.BlockSpec((None, F2, H), _slot(off)))
        w_specs.append(pl.BlockSpec((None, H, F), _slot(off)))

    grid_spec = pltpu.PrefetchScalarGridSpec(
        num_scalar_prefetch=2,
        grid=(E // SLOTS,),
        in_specs=[
            pl.BlockSpec((N, H), const2),
            pl.BlockSpec((N, H), const2),
            pl.BlockSpec((N, E), const2),
            pl.BlockSpec((E, F2), const2),
            pl.BlockSpec((E, H), const2),
        ] + w_specs,
        out_specs=pl.BlockSpec((N, H), const2),
        scratch_shapes=[pltpu.VMEM((F2, F), jnp.float32)],
    )

    out = pl.pallas_call(
        _ffn_body,
        grid_spec=grid_spec,
        out_shape=jax.ShapeDtypeStruct((N, H), jnp.float32),
        compiler_params=pltpu.CompilerParams(
            dimension_semantics=("arbitrary",)),
    )(sched.reshape(E), nact.reshape(1), t_norm, xt, ct, mlp1_b, mlp2_b,
      mlp1_w, mlp2_w, mlp1_w, mlp2_w, mlp1_w, mlp2_w, mlp1_w, mlp2_w)

    return out.reshape(B, T, H).astype(x.dtype)


# speedup vs baseline: 4.0713x; 1.0425x over previous
import jax
import jax.numpy as jnp
from jax.experimental import pallas as pl
from jax.experimental.pallas import tpu as pltpu

ALPHA = 1.702
LIMIT = 7.0
EPS = 1e-05
NEG = -1e30
SLOTS = 4


def _router_body(x_ref, scale_ref, gw_ref, gb_ref,
                 t_ref, ct_ref, sched_ref, n_ref):
    x = x_ref[...]
    ms = jnp.mean(x * x, axis=1, keepdims=True)
    t = x * jax.lax.rsqrt(ms + EPS) * scale_ref[...]
    t_ref[...] = t
    g = jax.lax.dot_general(t, gw_ref[...], (((1,), (1,)), ((), ())),
                            preferred_element_type=jnp.float32)
    g = g + gb_ref[...]
    n_tok, n_exp = g.shape
    iota_e = jax.lax.broadcasted_iota(jnp.int32, (n_tok, n_exp), 1)
    v1 = jnp.max(g, axis=1, keepdims=True)
    idx1 = jnp.min(jnp.where(g == v1, iota_e, n_exp), axis=1, keepdims=True)
    e1 = iota_e == idx1
    g2 = jnp.where(e1, NEG, g)
    v2 = jnp.max(g2, axis=1, keepdims=True)
    idx2 = jnp.min(jnp.where(g2 == v2, iota_e, n_exp), axis=1, keepdims=True)
    e2 = iota_e == idx2
    p1 = jax.nn.sigmoid(v1 - v2)
    p2 = 1.0 - p1
    ct_ref[...] = p1 * e1.astype(jnp.float32) + p2 * e2.astype(jnp.float32)
    hits = jnp.transpose((e1 | e2).astype(jnp.float32))
    cnts = jnp.sum(hits, axis=1, keepdims=True)
    m = (cnts > 0.0).astype(jnp.float32)
    row = jax.lax.broadcasted_iota(jnp.int32, (n_exp, n_exp), 0).astype(jnp.float32)
    col = jax.lax.broadcasted_iota(jnp.int32, (n_exp, n_exp), 1).astype(jnp.float32)
    tril = (row >= col).astype(jnp.float32)
    c = jax.lax.dot_general(tril, m, (((1,), (0,)), ((), ())),
                            preferred_element_type=jnp.float32)
    onehot = m * (c - 1.0 == col).astype(jnp.float32)
    sched = jnp.sum(row * onehot, axis=0, keepdims=True)
    sched_ref[...] = sched.astype(jnp.int32)
    n_ref[...] = jnp.sum(m, keepdims=True).astype(jnp.int32)


def _expert_contrib(t, ct, valid, e, w1t_ref, w1b_ref, w2_ref,
                    b1_ref, b2_ref, p_ref):
    ht = jax.lax.dot_general(t, w1t_ref[...], (((1,), (1,)), ((), ())),
                             preferred_element_type=jnp.float32)
    hb = jax.lax.dot_general(t, w1b_ref[...], (((1,), (1,)), ((), ())),
                             preferred_element_type=jnp.float32)
    h = jnp.concatenate([ht, hb], axis=1)
    h = h + b1_ref[pl.ds(e, 1), :]
    a = jnp.minimum(h, LIMIT)
    s = (a * jax.nn.sigmoid(ALPHA * a)) * (LIMIT + 1.0)
    sc = jax.lax.dot_general(s, p_ref[...], (((1,), (0,)), ((), ())),
                             preferred_element_type=jnp.float32)
    h2 = jax.lax.dot_general(sc, w2_ref[...], (((1,), (1,)), ((), ())),
                             preferred_element_type=jnp.float32)
    h2 = h2 + b2_ref[pl.ds(e, 1), :]
    lane = jax.lax.broadcasted_iota(jnp.int32, ct.shape, 1)
    keep = jnp.logical_and(lane == e, valid)
    cvec = jnp.sum(jnp.where(keep, ct, 0.0), axis=1, keepdims=True)
    return cvec * h2


def _ffn_body(sched_ref, n_ref, t_ref, x_ref, ct_ref, b1_ref, b2_ref,
              w1at_ref, w1ab_ref, w2a_ref, w1bt_ref, w1bb_ref, w2b_ref,
              w1ct_ref, w1cb_ref, w2c_ref, w1dt_ref, w1db_ref, w2d_ref,
              out_ref, p_ref):
    i = pl.program_id(0)
    n = n_ref[0]

    @pl.when(i == 0)
    def _init():
        out_ref[...] = x_ref[...]
        f2, f = p_ref.shape
        rr = jax.lax.broadcasted_iota(jnp.int32, (f2, f), 0)
        cc = jax.lax.broadcasted_iota(jnp.int32, (f2, f), 1)
        p_ref[...] = (rr == 2 * cc).astype(jnp.float32)

    @pl.when(SLOTS * i < n)
    def _compute():
        t = t_ref[...]
        ct = ct_ref[...]
        ws = [(w1at_ref, w1ab_ref, w2a_ref), (w1bt_ref, w1bb_ref, w2b_ref),
              (w1ct_ref, w1cb_ref, w2c_ref), (w1dt_ref, w1db_ref, w2d_ref)]
        acc = out_ref[...]
        for j, (w1t_ref, w1b_ref, w2_ref) in enumerate(ws):
            p = SLOTS * i + j
            e = sched_ref[jnp.minimum(p, n - 1)]
            valid = p < n if j else True
            acc = acc + _expert_contrib(t, ct, valid, e, w1t_ref, w1b_ref,
                                        w2_ref, b1_ref, b2_ref, p_ref)
        out_ref[...] = acc


def kernel(x, scale, gate_w, gate_b, mlp1_w, mlp1_b, mlp2_w, mlp2_b):
    B, T, H = x.shape
    N = B * T
    E, F2, _ = mlp1_w.shape
    F = F2 // 2
    xt = x.reshape(N, H).astype(jnp.float32)

    t_norm, ct, sched, nact = pl.pallas_call(
        _router_body,
        out_shape=(
            jax.ShapeDtypeStruct((N, H), jnp.float32),
            jax.ShapeDtypeStruct((N, E), jnp.float32),
            jax.ShapeDtypeStruct((1, E), jnp.int32),
            jax.ShapeDtypeStruct((1, 1), jnp.int32),
        ),
    )(xt, scale.reshape(1, H), gate_w, gate_b.reshape(1, E))

    def _slot(off):
        def _map(i, s, n):
            return (s[jnp.minimum(SLOTS * i + off, n[0] - 1)], 0, 0)
        return _map

    def _slot3(off, half):
        def _map(i, s, n):
            return (s[jnp.minimum(SLOTS * i + off, n[0] - 1)], half, 0)
        return _map

    const2 = lambda i, s, n: (0, 0)
    w_specs = []
    for off in range(SLOTS):
        w_specs.append(pl.BlockSpec((None, F, H), _slot3(off, 0)))
        w_specs.append(pl.BlockSpec((None, F, H), _slot3(off, 1)))
        w_specs.append(pl.BlockSpec((None, H, F), _slot(off)))

    grid_spec = pltpu.PrefetchScalarGridSpec(
        num_scalar_prefetch=2,
        grid=(E // SLOTS,),
        in_specs=[
            pl.BlockSpec((N, H), const2),
            pl.BlockSpec((N, H), const2),
            pl.BlockSpec((N, E), const2),
            pl.BlockSpec((E, F2), const2),
            pl.BlockSpec((E, H), const2),
        ] + w_specs,
        out_specs=pl.BlockSpec((N, H), const2),
        scratch_shapes=[pltpu.VMEM((F2, F), jnp.float32)],
    )

    out = pl.pallas_call(
        _ffn_body,
        grid_spec=grid_spec,
        out_shape=jax.ShapeDtypeStruct((N, H), jnp.float32),
        compiler_params=pltpu.CompilerParams(
            dimension_semantics=("arbitrary",)),
    )(sched.reshape(E), nact.reshape(1), t_norm, xt, ct, mlp1_b, mlp2_b,
      mlp1_w, mlp1_w, mlp2_w, mlp1_w, mlp1_w, mlp2_w,
      mlp1_w, mlp1_w, mlp2_w, mlp1_w, mlp1_w, mlp2_w)

    return out.reshape(B, T, H).astype(x.dtype)


# speedup vs baseline: 4.2690x; 1.0485x over previous
import jax
import jax.numpy as jnp
from jax.experimental import pallas as pl
from jax.experimental.pallas import tpu as pltpu

ALPHA = 1.702
LIMIT = 7.0
EPS = 1e-05
NEG = -1e30
SLOTS = 4


def _router_body(x_ref, scale_ref, gw_ref, gb_ref,
                 t_ref, ct_ref, sched_ref, n_ref):
    x3 = x_ref[...]
    x = x3.reshape(x3.shape[0] * x3.shape[1], x3.shape[2])
    ms = jnp.mean(x * x, axis=1, keepdims=True)
    t = x * jax.lax.rsqrt(ms + EPS) * scale_ref[...]
    t_ref[...] = t
    g = jax.lax.dot_general(t, gw_ref[...], (((1,), (1,)), ((), ())),
                            preferred_element_type=jnp.float32)
    g = g + gb_ref[...]
    n_tok, n_exp = g.shape
    iota_e = jax.lax.broadcasted_iota(jnp.int32, (n_tok, n_exp), 1)
    v1 = jnp.max(g, axis=1, keepdims=True)
    idx1 = jnp.min(jnp.where(g == v1, iota_e, n_exp), axis=1, keepdims=True)
    e1 = iota_e == idx1
    g2 = jnp.where(e1, NEG, g)
    v2 = jnp.max(g2, axis=1, keepdims=True)
    idx2 = jnp.min(jnp.where(g2 == v2, iota_e, n_exp), axis=1, keepdims=True)
    e2 = iota_e == idx2
    p1 = jax.nn.sigmoid(v1 - v2)
    p2 = 1.0 - p1
    ct_ref[...] = p1 * e1.astype(jnp.float32) + p2 * e2.astype(jnp.float32)
    hits = jnp.transpose((e1 | e2).astype(jnp.float32))
    cnts = jnp.sum(hits, axis=1, keepdims=True)
    m = (cnts > 0.0).astype(jnp.float32)
    row = jax.lax.broadcasted_iota(jnp.int32, (n_exp, n_exp), 0).astype(jnp.float32)
    col = jax.lax.broadcasted_iota(jnp.int32, (n_exp, n_exp), 1).astype(jnp.float32)
    tril = (row >= col).astype(jnp.float32)
    c = jax.lax.dot_general(tril, m, (((1,), (0,)), ((), ())),
                            preferred_element_type=jnp.float32)
    onehot = m * (c - 1.0 == col).astype(jnp.float32)
    sched = jnp.sum(row * onehot, axis=0, keepdims=True)
    sched_ref[...] = sched.astype(jnp.int32)
    n_ref[...] = jnp.sum(m, keepdims=True).astype(jnp.int32)


def _expert_contrib(t, ct, valid, e, w1t_ref, w1b_ref, w2_ref,
                    b1_ref, b2_ref, p_ref):
    ht = jax.lax.dot_general(t, w1t_ref[...], (((1,), (1,)), ((), ())),
                             preferred_element_type=jnp.float32)
    hb = jax.lax.dot_general(t, w1b_ref[...], (((1,), (1,)), ((), ())),
                             preferred_element_type=jnp.float32)
    h = jnp.concatenate([ht, hb], axis=1)
    h = h + b1_ref[pl.ds(e, 1), :]
    a = jnp.minimum(h, LIMIT)
    s = (a * jax.nn.sigmoid(ALPHA * a)) * (LIMIT + 1.0)
    sc = jax.lax.dot_general(s, p_ref[...], (((1,), (0,)), ((), ())),
                             preferred_element_type=jnp.float32)
    h2 = jax.lax.dot_general(sc, w2_ref[...], (((1,), (1,)), ((), ())),
                             preferred_element_type=jnp.float32)
    h2 = h2 + b2_ref[pl.ds(e, 1), :]
    lane = jax.lax.broadcasted_iota(jnp.int32, ct.shape, 1)
    keep = jnp.logical_and(lane == e, valid)
    cvec = jnp.sum(jnp.where(keep, ct, 0.0), axis=1, keepdims=True)
    return cvec * h2


def _ffn_body(sched_ref, n_ref, t_ref, x_ref, ct_ref, b1_ref, b2_ref,
              w1at_ref, w1ab_ref, w2a_ref, w1bt_ref, w1bb_ref, w2b_ref,
              w1ct_ref, w1cb_ref, w2c_ref, w1dt_ref, w1db_ref, w2d_ref,
              out_ref, p_ref):
    i = pl.program_id(0)
    n = n_ref[0, 0]

    @pl.when(i == 0)
    def _init():
        out_ref[...] = x_ref[...]
        f2, f = p_ref.shape
        rr = jax.lax.broadcasted_iota(jnp.int32, (f2, f), 0)
        cc = jax.lax.broadcasted_iota(jnp.int32, (f2, f), 1)
        p_ref[...] = (rr == 2 * cc).astype(jnp.float32)

    @pl.when(SLOTS * i < n)
    def _compute():
        t = t_ref[...]
        ct = ct_ref[...]
        ws = [(w1at_ref, w1ab_ref, w2a_ref), (w1bt_ref, w1bb_ref, w2b_ref),
              (w1ct_ref, w1cb_ref, w2c_ref), (w1dt_ref, w1db_ref, w2d_ref)]
        o3 = out_ref[...]
        acc = o3.reshape(t.shape)
        for j, (w1t_ref, w1b_ref, w2_ref) in enumerate(ws):
            p = SLOTS * i + j
            e = sched_ref[0, jnp.minimum(p, n - 1)]
            valid = p < n if j else True
            acc = acc + _expert_contrib(t, ct, valid, e, w1t_ref, w1b_ref,
                                        w2_ref, b1_ref, b2_ref, p_ref)
        out_ref[...] = acc.reshape(o3.shape)


def kernel(x, scale, gate_w, gate_b, mlp1_w, mlp1_b, mlp2_w, mlp2_b):
    B, T, H = x.shape
    N = B * T
    E, F2, _ = mlp1_w.shape
    F = F2 // 2
    xf = x.astype(jnp.float32)

    t_norm, ct, sched, nact = pl.pallas_call(
        _router_body,
        out_shape=(
            jax.ShapeDtypeStruct((N, H), jnp.float32),
            jax.ShapeDtypeStruct((N, E), jnp.float32),
            jax.ShapeDtypeStruct((1, E), jnp.int32),
            jax.ShapeDtypeStruct((1, 1), jnp.int32),
        ),
    )(xf, scale.reshape(1, H), gate_w, gate_b.reshape(1, E))

    def _slot(off):
        def _map(i, s, n):
            return (s[0, jnp.minimum(SLOTS * i + off, n[0, 0] - 1)], 0, 0)
        return _map

    def _slot3(off, half):
        def _map(i, s, n):
            return (s[0, jnp.minimum(SLOTS * i + off, n[0, 0] - 1)], half, 0)
        return _map

    const2 = lambda i, s, n: (0, 0)
    const3 = lambda i, s, n: (0, 0, 0)
    w_specs = []
    for off in range(SLOTS):
        w_specs.append(pl.BlockSpec((None, F, H), _slot3(off, 0)))
        w_specs.append(pl.BlockSpec((None, F, H), _slot3(off, 1)))
        w_specs.append(pl.BlockSpec((None, H, F), _slot(off)))

    grid_spec = pltpu.PrefetchScalarGridSpec(
        num_scalar_prefetch=2,
        grid=(E // SLOTS,),
        in_specs=[
            pl.BlockSpec((N, H), const2),
            pl.BlockSpec((B, T, H), const3),
            pl.BlockSpec((N, E), const2),
            pl.BlockSpec((E, F2), const2),
            pl.BlockSpec((E, H), const2),
        ] + w_specs,
        out_specs=pl.BlockSpec((B, T, H), const3),
        scratch_shapes=[pltpu.VMEM((F2, F), jnp.float32)],
    )

    out = pl.pallas_call(
        _ffn_body,
        grid_spec=grid_spec,
        out_shape=jax.ShapeDtypeStruct((B, T, H), jnp.float32),
        compiler_params=pltpu.CompilerParams(
            dimension_semantics=("arbitrary",)),
    )(sched, nact, t_norm, xf, ct, mlp1_b, mlp2_b,
      mlp1_w, mlp1_w, mlp2_w, mlp1_w, mlp1_w, mlp2_w,
      mlp1_w, mlp1_w, mlp2_w, mlp1_w, mlp1_w, mlp2_w)

    return out.astype(x.dtype)
